# Initial kernel scaffold; baseline (speedup 1.0000x reference)
import jax
import jax.numpy as jnp
from jax.experimental import pallas as pl


def kernel(x, router_W, router_b, up_W, up_b, down_W, down_b):
    raise NotImplementedError("write your pallas kernel here")


# speedup vs baseline: 1.2693x; 1.2693x over previous
import functools

import jax
import jax.numpy as jnp
from jax import lax
from jax.experimental import pallas as pl
from jax.experimental.pallas import tpu as pltpu

NUM_EXPERTS = 8
TOP_K = 2
HIDDEN = 2048
D_MODEL = 1024
AUX_W = 0.01
EPAD = 128


def _router_body(x_ref, w_ref, b_ref, probs_ref, sel_ref, misc_ref):
    x = x_ref[...]
    W = w_ref[...]
    n = x.shape[0]
    logits = jnp.dot(x, W, preferred_element_type=jnp.float32) + b_ref[...]
    col = lax.broadcasted_iota(jnp.int32, (n, EPAD), 1)
    neg = jnp.where(col < NUM_EXPERTS, logits, -1e30)
    m = jnp.max(neg, axis=1, keepdims=True)
    ex = jnp.exp(neg - m)
    probs = ex / jnp.sum(ex, axis=1, keepdims=True)
    probs_ref[...] = probs
    m0 = jnp.max(neg, axis=1, keepdims=True)
    i0 = jnp.min(jnp.where(neg == m0, col, EPAD), axis=1, keepdims=True)
    neg1 = jnp.where(col == i0, -1e30, neg)
    m1 = jnp.max(neg1, axis=1, keepdims=True)
    i1 = jnp.min(jnp.where(neg1 == m1, col, EPAD), axis=1, keepdims=True)
    t = jnp.exp(m1 - m0)
    w0 = 1.0 / (1.0 + t)
    w1 = t / (1.0 + t)
    sel = jnp.where(col == 0, w0, 0.0)
    sel = sel + jnp.where(col == 1, w1, 0.0)
    sel = sel + jnp.where(col == 2, i0.astype(jnp.float32), 0.0)
    sel = sel + jnp.where(col == 3, i1.astype(jnp.float32), 0.0)
    sel_ref[...] = sel
    cnt = jnp.sum(jnp.where(col == i0, 1.0, 0.0) + jnp.where(col == i1, 1.0, 0.0),
                  axis=0, keepdims=True)
    pm = jnp.mean(probs, axis=0, keepdims=True)
    aux = jnp.sum(cnt * pm) * (NUM_EXPERTS * AUX_W / (n * TOP_K))
    row = lax.broadcasted_iota(jnp.int32, (8, EPAD), 0)
    misc = jnp.where(row == 0, jnp.broadcast_to(cnt, (8, EPAD)), 0.0)
    misc = misc + jnp.where(row == 1, jnp.broadcast_to(pm, (8, EPAD)), 0.0)
    misc = misc + jnp.where(row == 2, aux, 0.0)
    misc_ref[...] = misc


def _expert_body(x_ref, upw_ref, upb_ref, dnw_ref, dnb_ref, sel_ref, out_ref):
    e = pl.program_id(0)
    hc = pl.program_id(1)
    n = x_ref.shape[0]
    CH = 512

    @pl.when((e == 0) & (hc == 0))
    def _():
        out_ref[...] = jnp.zeros_like(out_ref)

    ef = e.astype(jnp.float32)
    upw = upw_ref[0]
    upb = upb_ref[0, 0]
    dnw = dnw_ref[0]
    dnb = dnb_ref[0, 0]

    def chunk(c, _):
        xc = x_ref[pl.ds(c * CH, CH), :]
        h = jnp.dot(xc, upw, preferred_element_type=jnp.float32) + upb[None, :]
        h = jax.nn.gelu(h)
        y = jnp.dot(h, dnw, preferred_element_type=jnp.float32)
        sc = sel_ref[pl.ds(c * CH, CH), :]
        col = lax.broadcasted_iota(jnp.int32, sc.shape, 1)
        w0 = jnp.sum(jnp.where(col == 0, sc, 0.0), axis=1, keepdims=True)
        w1 = jnp.sum(jnp.where(col == 1, sc, 0.0), axis=1, keepdims=True)
        i0 = jnp.sum(jnp.where(col == 2, sc, 0.0), axis=1, keepdims=True)
        i1 = jnp.sum(jnp.where(col == 3, sc, 0.0), axis=1, keepdims=True)
        wc = (w0 * (i0 == ef).astype(jnp.float32)
              + w1 * (i1 == ef).astype(jnp.float32))
        y = jnp.where(hc == 0, y + dnb[None, :], y)
        out_ref[pl.ds(c * CH, CH), :] += wc * y
        return 0

    lax.fori_loop(0, n // CH, chunk, 0)


def kernel(x, router_W, router_b, up_W, up_b, down_W, down_b):
    B, S, D = x.shape
    N = B * S
    x2 = x.reshape(N, D)
    Wp = jnp.zeros((D, EPAD), jnp.float32).at[:, :NUM_EXPERTS].set(router_W)
    bp = jnp.zeros((1, EPAD), jnp.float32).at[0, :NUM_EXPERTS].set(router_b)

    probs_p, sel, misc = pl.pallas_call(
        _router_body,
        out_shape=[
            jax.ShapeDtypeStruct((N, EPAD), jnp.float32),
            jax.ShapeDtypeStruct((N, EPAD), jnp.float32),
            jax.ShapeDtypeStruct((8, EPAD), jnp.float32),
        ],
        compiler_params=pltpu.CompilerParams(
            vmem_limit_bytes=100 * 1024 * 1024),
    )(x2, Wp, bp)

    HC = HIDDEN // 2
    out = pl.pallas_call(
        _expert_body,
        grid=(NUM_EXPERTS, 2),
        in_specs=[
            pl.BlockSpec((N, D), lambda e, h: (0, 0)),
            pl.BlockSpec((1, D, HC), lambda e, h: (e, 0, h)),
            pl.BlockSpec((1, 1, HC), lambda e, h: (e, 0, h)),
            pl.BlockSpec((1, HC, D), lambda e, h: (e, h, 0)),
            pl.BlockSpec((1, 1, D), lambda e, h: (e, 0, 0)),
            pl.BlockSpec((N, EPAD), lambda e, h: (0, 0)),
        ],
        out_specs=pl.BlockSpec((N, D), lambda e, h: (0, 0)),
        out_shape=jax.ShapeDtypeStruct((N, D), jnp.float32),
        compiler_params=pltpu.CompilerParams(
            dimension_semantics=("arbitrary", "arbitrary"),
            vmem_limit_bytes=63 * 1024 * 1024),
    )(x2, up_W, up_b.reshape(NUM_EXPERTS, 1, HIDDEN),
      down_W, down_b.reshape(NUM_EXPERTS, 1, D), sel)

    router_probs = probs_p[:, :NUM_EXPERTS].reshape(B, S, NUM_EXPERTS)
    aux_loss = misc[2, 0]
    return (out.reshape(B, S, D), aux_loss, router_probs)


# speedup vs baseline: 1.6504x; 1.3003x over previous
import functools

import jax
import jax.numpy as jnp
from jax import lax
from jax.experimental import pallas as pl
from jax.experimental.pallas import tpu as pltpu
from jax.experimental.pallas import tpu_sc as plsc

NUM_EXPERTS = 8
TOP_K = 2
HIDDEN = 2048
D_MODEL = 1024
AUX_W = 0.01
EPAD = 128
TILE = 128
N_TOK = 4096
N_GRID = (N_TOK * TOP_K) // TILE + NUM_EXPERTS
NPAD = N_GRID * TILE
CHUNK = 128

_sc_kernels_cache = {}


def _router_body(x_ref, w_ref, b_ref,
                 probs_ref, sel_ref, misc_ref, tmap_ref, w0b_ref, w1b_ref,
                 neg_s):
    n = x_ref.shape[0]
    nch = n // CHUNK
    x = x_ref[...]
    W = w_ref[...]
    logits = jnp.dot(x, W, preferred_element_type=jnp.float32) + b_ref[...]
    colf = lax.broadcasted_iota(jnp.int32, (n, EPAD), 1).astype(jnp.float32)
    neg_s[...] = jnp.where(colf < NUM_EXPERTS, logits, -1e30)

    cc = lax.broadcasted_iota(jnp.int32, (CHUNK, EPAD), 1).astype(jnp.float32)
    rr = lax.broadcasted_iota(jnp.int32, (CHUNK, EPAD), 0).astype(jnp.float32)
    ltri = (cc < rr).astype(jnp.float32)
    utri = (rr < cc).astype(jnp.float32)

    def phase1(c, carry):
        off0, off1, psum = carry
        neg = neg_s[pl.ds(c * CHUNK, CHUNK), :]
        m0 = jnp.max(neg, axis=1, keepdims=True)
        i0 = jnp.min(jnp.where(neg == m0, cc, 1e9), axis=1, keepdims=True)
        mask0 = (cc == i0).astype(jnp.float32)
        neg1 = jnp.where(cc == i0, -1e30, neg)
        m1 = jnp.max(neg1, axis=1, keepdims=True)
        i1 = jnp.min(jnp.where(neg1 == m1, cc, 1e9), axis=1, keepdims=True)
        mask1 = (cc == i1).astype(jnp.float32)
        ex = jnp.exp(neg - m0)
        probs = ex / jnp.sum(ex, axis=1, keepdims=True)
        probs_ref[pl.ds(c * CHUNK, CHUNK), :] = probs
        t = jnp.exp(m1 - m0)
        w0 = 1.0 / (1.0 + t)
        w1 = t / (1.0 + t)
        w0b_ref[pl.ds(c * CHUNK, CHUNK), :] = jnp.broadcast_to(w0, (CHUNK, EPAD))
        w1b_ref[pl.ds(c * CHUNK, CHUNK), :] = jnp.broadcast_to(w1, (CHUNK, EPAD))
        cum0 = jnp.dot(ltri, mask0, preferred_element_type=jnp.float32)
        cum1 = jnp.dot(ltri, mask1, preferred_element_type=jnp.float32)
        rank0 = (jnp.sum(cum0 * mask0, axis=1, keepdims=True)
                 + jnp.sum(off0 * mask0, axis=1, keepdims=True))
        rank1 = (jnp.sum(cum1 * mask1, axis=1, keepdims=True)
                 + jnp.sum(off1 * mask1, axis=1, keepdims=True))
        sel = jnp.where(cc == 0, w0, 0.0)
        sel = sel + jnp.where(cc == 1, w1, 0.0)
        sel = sel + jnp.where(cc == 2, i0, 0.0)
        sel = sel + jnp.where(cc == 3, i1, 0.0)
        sel = sel + jnp.where(cc == 4, rank0, 0.0)
        sel = sel + jnp.where(cc == 5, rank1, 0.0)
        sel_ref[pl.ds(c * CHUNK, CHUNK), :] = sel
        return (off0 + jnp.sum(mask0, axis=0, keepdims=True),
                off1 + jnp.sum(mask1, axis=0, keepdims=True),
                psum + jnp.sum(probs, axis=0, keepdims=True))

    zero = jnp.zeros((1, EPAD), jnp.float32)
    cnt0, cnt1, psum = lax.fori_loop(0, nch, phase1, (zero, zero, zero))

    cnt = cnt0 + cnt1
    pm = psum / n
    aux = jnp.sum(cnt * pm) * (NUM_EXPERTS * AUX_W / (n * TOP_K))
    pc = jnp.floor((cnt + (TILE - 1)) / TILE) * TILE
    padbase = jnp.dot(pc, utri, preferred_element_type=jnp.float32)

    row8 = lax.broadcasted_iota(jnp.int32, (8, EPAD), 0)
    misc = jnp.where(row8 == 0, jnp.broadcast_to(cnt, (8, EPAD)), 0.0)
    misc = misc + jnp.where(row8 == 1, jnp.broadcast_to(pm, (8, EPAD)), 0.0)
    misc = misc + jnp.where(row8 == 2, aux, 0.0)
    misc_ref[...] = misc

    pbb = jnp.broadcast_to(padbase, (CHUNK, EPAD))
    c0b = jnp.broadcast_to(cnt0, (CHUNK, EPAD))

    def phase2(c, _):
        sel = sel_ref[pl.ds(c * CHUNK, CHUNK), :]
        i0 = jnp.sum(jnp.where(cc == 2, sel, 0.0), axis=1, keepdims=True)
        i1 = jnp.sum(jnp.where(cc == 3, sel, 0.0), axis=1, keepdims=True)
        rank0 = jnp.sum(jnp.where(cc == 4, sel, 0.0), axis=1, keepdims=True)
        rank1 = jnp.sum(jnp.where(cc == 5, sel, 0.0), axis=1, keepdims=True)
        pb0 = jnp.sum(jnp.where(cc == i0, pbb, 0.0), axis=1, keepdims=True)
        pb1 = jnp.sum(jnp.where(cc == i1, pbb, 0.0), axis=1, keepdims=True)
        c01 = jnp.sum(jnp.where(cc == i1, c0b, 0.0), axis=1, keepdims=True)
        pos0 = pb0 + rank0
        pos1 = pb1 + c01 + rank1
        sel2 = (jnp.where(cc == 4, pos0 - rank0, 0.0)
                + jnp.where(cc == 5, pos1 - rank1, 0.0))
        sel_ref[pl.ds(c * CHUNK, CHUNK), :] = sel + sel2
        return 0

    lax.fori_loop(0, nch, phase2, 0)

    gi = lax.broadcasted_iota(jnp.int32, (CHUNK, EPAD), 0).astype(jnp.float32)
    ce = lax.broadcasted_iota(jnp.int32, (CHUNK, EPAD), 1).astype(jnp.float32)
    ende = jnp.broadcast_to(padbase + pc, (CHUNK, EPAD))
    ind = ((gi * TILE >= ende) & (ce < NUM_EXPERTS)).astype(jnp.int32)
    te = jnp.sum(ind, axis=1, keepdims=True)
    te = jnp.minimum(te, NUM_EXPERTS - 1)
    tmap_ref[...] = jnp.broadcast_to(te, (CHUNK, EPAD))


def _dispatch_body(x_hbm, p0_hbm, p1_hbm, xs_hbm, xb, idx0, idx1, s0, s1):
    wid = lax.axis_index("s") * 2 + lax.axis_index("c")
    base = wid * 128
    for sub in range(2):
        tok = pl.multiple_of(base + sub * 64, 64)
        pltpu.sync_copy(p0_hbm.at[pl.ds(tok, 64)], idx0)
        pltpu.sync_copy(p1_hbm.at[pl.ds(tok, 64)], idx1)
        pltpu.sync_copy(x_hbm.at[pl.ds(tok, 64)], xb)
        c0 = pltpu.async_copy(xb, xs_hbm.at[idx0], s0)
        c1 = pltpu.async_copy(xb, xs_hbm.at[idx1], s1)
        c0.wait()
        c1.wait()


def _gmm_body(te_ref, xs_ref, upw_ref, upb_ref, dnw_ref, dnb_ref, out_ref):
    x = xs_ref[...]
    h = jnp.dot(x, upw_ref[0], preferred_element_type=jnp.float32)
    h = jax.nn.gelu(h + upb_ref[0, 0][None, :])
    y = jnp.dot(h, dnw_ref[0], preferred_element_type=jnp.float32)
    out_ref[...] = y + dnb_ref[0, 0][None, :]


def _combine_body(ys_hbm, p0_hbm, p1_hbm, w0b_hbm, w1b_hbm, out_hbm,
                  y0, y1, ob, idx0, idx1, w0v, w1v, s0, s1):
    wid = lax.axis_index("s") * 2 + lax.axis_index("c")
    base = wid * 128

    def chunk(ch, _):
        tok = pl.multiple_of(base + ch * 16, 16)
        pltpu.sync_copy(p0_hbm.at[pl.ds(tok, 16)], idx0)
        pltpu.sync_copy(p1_hbm.at[pl.ds(tok, 16)], idx1)
        c0 = pltpu.async_copy(ys_hbm.at[idx0], y0, s0)
        c1 = pltpu.async_copy(ys_hbm.at[idx1], y1, s1)
        pltpu.sync_copy(w0b_hbm.at[pl.ds(tok, 16)], w0v)
        pltpu.sync_copy(w1b_hbm.at[pl.ds(tok, 16)], w1v)
        c0.wait()
        c1.wait()

        def row(r, _):
            w0s = w0v[r, pl.ds(0, 16)]
            w1s = w1v[r, pl.ds(0, 16)]
            for v in range(D_MODEL // 16):
                ob[r, pl.ds(v * 16, 16)] = (
                    w0s * y0[r, pl.ds(v * 16, 16)]
                    + w1s * y1[r, pl.ds(v * 16, 16)])
            return 0

        lax.fori_loop(0, 16, row, 0)
        pltpu.sync_copy(ob, out_hbm.at[pl.ds(tok, 16)])
        return 0

    lax.fori_loop(0, 8, chunk, 0)


def _get_sc_kernels():
    if "k" not in _sc_kernels_cache:
        mesh = plsc.VectorSubcoreMesh(core_axis_name="c", subcore_axis_name="s")
        dispatch = pl.kernel(
            _dispatch_body, mesh=mesh,
            out_type=jax.ShapeDtypeStruct((NPAD, D_MODEL), jnp.float32),
            scratch_types=[
                pltpu.VMEM((64, D_MODEL), jnp.float32),
                pltpu.VMEM((64,), jnp.int32),
                pltpu.VMEM((64,), jnp.int32),
                pltpu.SemaphoreType.DMA,
                pltpu.SemaphoreType.DMA,
            ])
        combine = pl.kernel(
            _combine_body, mesh=mesh,
            out_type=jax.ShapeDtypeStruct((N_TOK, D_MODEL), jnp.float32),
            scratch_types=[
                pltpu.VMEM((16, D_MODEL), jnp.float32),
                pltpu.VMEM((16, D_MODEL), jnp.float32),
                pltpu.VMEM((16, D_MODEL), jnp.float32),
                pltpu.VMEM((16,), jnp.int32),
                pltpu.VMEM((16,), jnp.int32),
                pltpu.VMEM((16, EPAD), jnp.float32),
                pltpu.VMEM((16, EPAD), jnp.float32),
                pltpu.SemaphoreType.DMA,
                pltpu.SemaphoreType.DMA,
            ])
        _sc_kernels_cache["k"] = (dispatch, combine)
    return _sc_kernels_cache["k"]


def kernel(x, router_W, router_b, up_W, up_b, down_W, down_b):
    _dispatch, _combine = _get_sc_kernels()
    B, S, D = x.shape
    N = B * S
    x2 = x.reshape(N, D)
    Wp = jnp.zeros((D, EPAD), jnp.float32).at[:, :NUM_EXPERTS].set(router_W)
    bp = jnp.zeros((1, EPAD), jnp.float32).at[0, :NUM_EXPERTS].set(router_b)

    probs_p, sel, misc, tmap, w0b, w1b = pl.pallas_call(
        _router_body,
        out_shape=[
            jax.ShapeDtypeStruct((N, EPAD), jnp.float32),
            jax.ShapeDtypeStruct((N, EPAD), jnp.float32),
            jax.ShapeDtypeStruct((8, EPAD), jnp.float32),
            jax.ShapeDtypeStruct((CHUNK, EPAD), jnp.int32),
            jax.ShapeDtypeStruct((N, EPAD), jnp.float32),
            jax.ShapeDtypeStruct((N, EPAD), jnp.float32),
        ],
        scratch_shapes=[pltpu.VMEM((N, EPAD), jnp.float32)],
        compiler_params=pltpu.CompilerParams(
            vmem_limit_bytes=63 * 1024 * 1024),
    )(x2, Wp, bp)

    pos0 = sel[:, 4].astype(jnp.int32)
    pos1 = sel[:, 5].astype(jnp.int32)
    te = tmap[:N_GRID, 0]

    xs = _dispatch(x2, pos0, pos1)

    ys = pl.pallas_call(
        _gmm_body,
        grid_spec=pltpu.PrefetchScalarGridSpec(
            num_scalar_prefetch=1,
            grid=(N_GRID,),
            in_specs=[
                pl.BlockSpec((TILE, D), lambda g, te: (g, 0)),
                pl.BlockSpec((1, D, HIDDEN), lambda g, te: (te[g], 0, 0)),
                pl.BlockSpec((1, 1, HIDDEN), lambda g, te: (te[g], 0, 0)),
                pl.BlockSpec((1, HIDDEN, D), lambda g, te: (te[g], 0, 0)),
                pl.BlockSpec((1, 1, D), lambda g, te: (te[g], 0, 0)),
            ],
            out_specs=pl.BlockSpec((TILE, D), lambda g, te: (g, 0)),
        ),
        out_shape=jax.ShapeDtypeStruct((NPAD, D), jnp.float32),
        compiler_params=pltpu.CompilerParams(
            dimension_semantics=("arbitrary",),
            vmem_limit_bytes=63 * 1024 * 1024),
    )(te, xs, up_W, up_b.reshape(NUM_EXPERTS, 1, HIDDEN),
      down_W, down_b.reshape(NUM_EXPERTS, 1, D))

    out2 = _combine(ys, pos0, pos1, w0b, w1b)

    router_probs = probs_p[:, :NUM_EXPERTS].reshape(B, S, NUM_EXPERTS)
    aux_loss = misc[2, 0]
    return (out2.reshape(B, S, D), aux_loss, router_probs)
